# baseline (speedup 1.0000x reference)
import functools

import jax
import jax.numpy as jnp
from jax import lax
from jax.experimental import pallas as pl
from jax.experimental.pallas import tpu as pltpu
from jax.experimental.pallas import tpu_sc as plsc

_NUM_WORKERS = 32
_CHUNK = 3200
_NBUF = 2


@functools.partial(jax.jit, static_argnums=(2,))
def _sc_lookup(vocab_map, flat_tokens, n_per_worker):
    n_chunks = n_per_worker // _CHUNK
    mesh = plsc.VectorSubcoreMesh(
        core_axis_name="c", subcore_axis_name="s", num_cores=2, num_subcores=16
    )

    @functools.partial(
        pl.kernel,
        out_type=jax.ShapeDtypeStruct(flat_tokens.shape, jnp.float32),
        mesh=mesh,
        scratch_types=[
            pltpu.VMEM_SHARED(vocab_map.shape, jnp.float32),
            [pltpu.VMEM((_CHUNK,), jnp.int32) for _ in range(_NBUF)],
            [pltpu.VMEM((_CHUNK,), jnp.float32) for _ in range(_NBUF)],
            pltpu.SemaphoreType.DMA,
            [pltpu.SemaphoreType.DMA for _ in range(_NBUF)],
            pltpu.SemaphoreType.DMA,
            [pltpu.SemaphoreType.DMA for _ in range(_NBUF)],
        ],
        compiler_params=pltpu.CompilerParams(
            use_tc_tiling_on_sc=False, needs_layout_passes=False
        ),
    )
    def body(table_hbm, tok_hbm, out_hbm, table_sh, idx_v, out_v,
             sem_tab, sem_in, sem_g, sem_out):
        sid = lax.axis_index("s")
        wid = sid * 2 + lax.axis_index("c")
        base = wid * n_per_worker

        in_cps = [None] * _NBUF
        out_cps = [None] * _NBUF
        for c in range(min(_NBUF, n_chunks)):
            in_cps[c] = pltpu.async_copy(
                tok_hbm.at[pl.ds(base + c * _CHUNK, _CHUNK)],
                idx_v[c], sem_in[c],
            )

        @pl.when(sid == 0)
        def _():
            pltpu.async_copy(table_hbm, table_sh, sem_tab).wait()

        plsc.subcore_barrier()

        for c in range(n_chunks):
            b = c % _NBUF
            in_cps[b].wait()
            if out_cps[b] is not None:
                out_cps[b].wait()
            pltpu.async_copy(table_sh.at[idx_v[b]], out_v[b], sem_g).wait()
            out_cps[b] = pltpu.async_copy(
                out_v[b], out_hbm.at[pl.ds(base + c * _CHUNK, _CHUNK)],
                sem_out[b],
            )
            nxt = c + _NBUF
            if nxt < n_chunks:
                in_cps[b] = pltpu.async_copy(
                    tok_hbm.at[pl.ds(base + nxt * _CHUNK, _CHUNK)],
                    idx_v[b], sem_in[b],
                )
        for b in range(min(_NBUF, n_chunks)):
            if out_cps[b] is not None:
                out_cps[b].wait()

    return body(vocab_map, flat_tokens)


def kernel(tokens, start_idxs, end_idxs, vocab_map):
    b, s = tokens.shape
    n = b * s
    token_ids = _sc_lookup(vocab_map, tokens.reshape(n), n // _NUM_WORKERS)
    return token_ids.reshape(b, s), start_idxs, end_idxs


# speedup vs baseline: 1.0010x; 1.0010x over previous
import functools

import jax
import jax.numpy as jnp
from jax import lax
from jax.experimental import pallas as pl
from jax.experimental.pallas import tpu as pltpu
from jax.experimental.pallas import tpu_sc as plsc

_NUM_WORKERS = 32
_CHUNK = 6400
_NBUF = 4


@functools.partial(jax.jit, static_argnums=(2,))
def _sc_lookup(vocab_map, flat_tokens, n_per_worker):
    n_chunks = n_per_worker // _CHUNK
    mesh = plsc.VectorSubcoreMesh(
        core_axis_name="c", subcore_axis_name="s", num_cores=2, num_subcores=16
    )

    @functools.partial(
        pl.kernel,
        out_type=jax.ShapeDtypeStruct(flat_tokens.shape, jnp.float32),
        mesh=mesh,
        scratch_types=[
            pltpu.VMEM_SHARED(vocab_map.shape, jnp.float32),
            [pltpu.VMEM((_CHUNK,), jnp.int32) for _ in range(_NBUF)],
            [pltpu.VMEM((_CHUNK,), jnp.float32) for _ in range(_NBUF)],
            pltpu.SemaphoreType.DMA,
            [pltpu.SemaphoreType.DMA for _ in range(_NBUF)],
            pltpu.SemaphoreType.DMA,
            [pltpu.SemaphoreType.DMA for _ in range(_NBUF)],
        ],
        compiler_params=pltpu.CompilerParams(
            use_tc_tiling_on_sc=False, needs_layout_passes=False
        ),
    )
    def body(table_hbm, tok_hbm, out_hbm, table_sh, idx_v, out_v,
             sem_tab, sem_in, sem_g, sem_out):
        sid = lax.axis_index("s")
        wid = sid * 2 + lax.axis_index("c")
        base = wid * n_per_worker

        in_cps = [None] * _NBUF
        out_cps = [None] * _NBUF
        for c in range(min(_NBUF, n_chunks)):
            in_cps[c] = pltpu.async_copy(
                tok_hbm.at[pl.ds(base + c * _CHUNK, _CHUNK)],
                idx_v[c], sem_in[c],
            )

        @pl.when(sid == 0)
        def _():
            pltpu.async_copy(table_hbm, table_sh, sem_tab).wait()

        plsc.subcore_barrier()

        for c in range(n_chunks):
            b = c % _NBUF
            in_cps[b].wait()
            pltpu.async_copy(table_sh.at[idx_v[b]], out_v[b], sem_g).wait()
            out_cps[b] = pltpu.async_copy(
                out_v[b], out_hbm.at[pl.ds(base + c * _CHUNK, _CHUNK)],
                sem_out[b],
            )
        for b in range(min(_NBUF, n_chunks)):
            if out_cps[b] is not None:
                out_cps[b].wait()

    return body(vocab_map, flat_tokens)


def kernel(tokens, start_idxs, end_idxs, vocab_map):
    b, s = tokens.shape
    n = b * s
    token_ids = _sc_lookup(vocab_map, tokens.reshape(n), n // _NUM_WORKERS)
    return token_ids.reshape(b, s), start_idxs, end_idxs


# speedup vs baseline: 1.0027x; 1.0017x over previous
import functools

import jax
import jax.numpy as jnp
from jax import lax
from jax.experimental import pallas as pl
from jax.experimental.pallas import tpu as pltpu
from jax.experimental.pallas import tpu_sc as plsc

_NUM_WORKERS = 32
_CHUNK = 6400
_NBUF = 4


@functools.partial(jax.jit, static_argnums=(2,))
def _sc_lookup(vocab_map, flat_tokens, n_per_worker):
    n_chunks = n_per_worker // _CHUNK
    mesh = plsc.VectorSubcoreMesh(
        core_axis_name="c", subcore_axis_name="s", num_cores=2, num_subcores=16
    )

    @functools.partial(
        pl.kernel,
        out_type=jax.ShapeDtypeStruct(flat_tokens.shape, jnp.float32),
        mesh=mesh,
        scratch_types=[
            pltpu.VMEM_SHARED(vocab_map.shape, jnp.float32),
            [pltpu.VMEM((_CHUNK,), jnp.int32) for _ in range(_NBUF)],
            [pltpu.VMEM((_CHUNK,), jnp.float32) for _ in range(_NBUF)],
            pltpu.SemaphoreType.DMA,
            [pltpu.SemaphoreType.DMA for _ in range(_NBUF)],
            [pltpu.SemaphoreType.DMA for _ in range(_NBUF)],
            [pltpu.SemaphoreType.DMA for _ in range(_NBUF)],
        ],
        compiler_params=pltpu.CompilerParams(
            use_tc_tiling_on_sc=False, needs_layout_passes=False
        ),
    )
    def body(table_hbm, tok_hbm, out_hbm, table_sh, idx_v, out_v,
             sem_tab, sem_in, sem_g, sem_out):
        sid = lax.axis_index("s")
        wid = sid * 2 + lax.axis_index("c")
        base = wid * n_per_worker

        in_cps = [None] * _NBUF
        out_cps = [None] * _NBUF
        for c in range(min(_NBUF, n_chunks)):
            in_cps[c] = pltpu.async_copy(
                tok_hbm.at[pl.ds(base + c * _CHUNK, _CHUNK)],
                idx_v[c], sem_in[c],
            )

        @pl.when(sid == 0)
        def _():
            pltpu.async_copy(table_hbm, table_sh, sem_tab).wait()

        plsc.subcore_barrier()

        g_cps = [None] * _NBUF
        for c in range(n_chunks):
            in_cps[c].wait()
            g_cps[c] = pltpu.async_copy(
                table_sh.at[idx_v[c]], out_v[c], sem_g[c]
            )
        for c in range(n_chunks):
            g_cps[c].wait()
            out_cps[c] = pltpu.async_copy(
                out_v[c], out_hbm.at[pl.ds(base + c * _CHUNK, _CHUNK)],
                sem_out[c],
            )
        for c in range(n_chunks):
            out_cps[c].wait()

    return body(vocab_map, flat_tokens)


def kernel(tokens, start_idxs, end_idxs, vocab_map):
    b, s = tokens.shape
    n = b * s
    token_ids = _sc_lookup(vocab_map, tokens.reshape(n), n // _NUM_WORKERS)
    return token_ids.reshape(b, s), start_idxs, end_idxs


# speedup vs baseline: 1.0095x; 1.0068x over previous
import functools

import jax
import jax.numpy as jnp
from jax import lax
from jax.experimental import pallas as pl
from jax.experimental.pallas import tpu as pltpu
from jax.experimental.pallas import tpu_sc as plsc

_NUM_WORKERS = 32
_CHUNK = 6400
_NBUF = 2


@functools.partial(jax.jit, static_argnums=(2,))
def _sc_lookup(vocab_map, flat_tokens, n_per_worker):
    n_chunks = n_per_worker // _CHUNK
    mesh = plsc.VectorSubcoreMesh(
        core_axis_name="c", subcore_axis_name="s", num_cores=2, num_subcores=16
    )

    @functools.partial(
        pl.kernel,
        out_type=jax.ShapeDtypeStruct(flat_tokens.shape, jnp.float32),
        mesh=mesh,
        scratch_types=[
            pltpu.VMEM_SHARED(vocab_map.shape, jnp.float32),
            [pltpu.VMEM((_CHUNK,), jnp.int32) for _ in range(_NBUF)],
            [pltpu.VMEM((_CHUNK,), jnp.float32) for _ in range(_NBUF)],
            pltpu.SemaphoreType.DMA,
            [pltpu.SemaphoreType.DMA for _ in range(_NBUF)],
            pltpu.SemaphoreType.DMA,
            [pltpu.SemaphoreType.DMA for _ in range(_NBUF)],
        ],
        compiler_params=pltpu.CompilerParams(
            use_tc_tiling_on_sc=False, needs_layout_passes=False
        ),
    )
    def body(table_hbm, tok_hbm, out_hbm, table_sh, idx_v, out_v,
             sem_tab, sem_in, sem_g, sem_out):
        sid = lax.axis_index("s")
        wid = sid * 2 + lax.axis_index("c")
        base = wid * n_per_worker

        in_cps = [None] * _NBUF
        out_cps = [None] * _NBUF
        for c in range(min(_NBUF, n_chunks)):
            in_cps[c] = pltpu.async_copy(
                tok_hbm.at[pl.ds(base + c * _CHUNK, _CHUNK)],
                idx_v[c], sem_in[c],
            )

        @pl.when(sid == 0)
        def _():
            pltpu.async_copy(table_hbm, table_sh, sem_tab).wait()

        plsc.subcore_barrier()

        for c in range(n_chunks):
            b = c % _NBUF
            in_cps[b].wait()
            if out_cps[b] is not None:
                out_cps[b].wait()
            pltpu.async_copy(table_sh.at[idx_v[b]], out_v[b], sem_g).wait()
            out_cps[b] = pltpu.async_copy(
                out_v[b], out_hbm.at[pl.ds(base + c * _CHUNK, _CHUNK)],
                sem_out[b],
            )
            nxt = c + _NBUF
            if nxt < n_chunks:
                in_cps[b] = pltpu.async_copy(
                    tok_hbm.at[pl.ds(base + nxt * _CHUNK, _CHUNK)],
                    idx_v[b], sem_in[b],
                )
        for b in range(min(_NBUF, n_chunks)):
            if out_cps[b] is not None:
                out_cps[b].wait()

    return body(vocab_map, flat_tokens)


def kernel(tokens, start_idxs, end_idxs, vocab_map):
    b, s = tokens.shape
    n = b * s
    token_ids = _sc_lookup(vocab_map, tokens.reshape(n), n // _NUM_WORKERS)
    return token_ids.reshape(b, s), start_idxs, end_idxs
